# Initial kernel scaffold; baseline (speedup 1.0000x reference)
import jax
import jax.numpy as jnp
from jax.experimental import pallas as pl


def kernel(x, table):
    raise NotImplementedError("write your pallas kernel here")


# speedup vs baseline: 4.0418x; 4.0418x over previous
import functools
import math

import jax
import jax.numpy as jnp
from jax import lax
from jax.experimental import pallas as pl
from jax.experimental.pallas import tpu as pltpu
from jax.experimental.pallas import tpu_sc as plsc

EMB_DEPTH = 32
ROWS = 16384
COLS = 200
B = ROWS * COLS
SCALE = math.sqrt(float(EMB_DEPTH))

_info = plsc.get_sparse_core_info()
NC = _info.num_cores
NS = _info.num_subcores
L = _info.num_lanes
NW = NC * NS
BPW = B // NW
CHUNK = 1024
NCHUNK = BPW // CHUNK

assert B % NW == 0 and BPW % CHUNK == 0 and CHUNK % 8 == 0

_mesh = plsc.VectorSubcoreMesh(core_axis_name="c", subcore_axis_name="s")


@functools.partial(
    pl.kernel,
    mesh=_mesh,
    out_type=jax.ShapeDtypeStruct((B, EMB_DEPTH), jnp.float32),
    scratch_types=[
        pltpu.VMEM((CHUNK,), jnp.int32),
        pltpu.VMEM((CHUNK, EMB_DEPTH), jnp.float32),
        pltpu.SemaphoreType.DMA,
    ],
    compiler_params=pltpu.CompilerParams(use_tc_tiling_on_sc=False),
)
def _embed(x_hbm, table_hbm, out_hbm, idx_v, rows_v, sem):
    wid = lax.axis_index("s") * NC + lax.axis_index("c")
    base = wid * BPW

    def chunk_body(g, carry):
        off = base + g * CHUNK
        pltpu.sync_copy(x_hbm.at[pl.ds(off, CHUNK)], idx_v)
        pltpu.async_copy(table_hbm.at[idx_v], rows_v, sem).wait()

        def row_body(i, c):
            rows_v[i, pl.ds(0, L)] = rows_v[i, pl.ds(0, L)] * SCALE
            rows_v[i, pl.ds(L, L)] = rows_v[i, pl.ds(L, L)] * SCALE
            return c

        lax.fori_loop(0, CHUNK, row_body, 0)
        pltpu.sync_copy(rows_v, out_hbm.at[pl.ds(off, CHUNK)])
        return carry

    lax.fori_loop(0, NCHUNK, chunk_body, 0)


def kernel(x, table):
    out = _embed(x.reshape(B), table)
    return out.reshape(ROWS, COLS, EMB_DEPTH)


# speedup vs baseline: 5.0056x; 1.2384x over previous
import functools
import math

import jax
import jax.numpy as jnp
from jax import lax
from jax.experimental import pallas as pl
from jax.experimental.pallas import tpu as pltpu
from jax.experimental.pallas import tpu_sc as plsc

EMB_DEPTH = 32
ROWS = 16384
COLS = 200
B = ROWS * COLS
SCALE = math.sqrt(float(EMB_DEPTH))

_info = plsc.get_sparse_core_info()
NC = _info.num_cores
NS = _info.num_subcores
L = _info.num_lanes
NW = NC * NS
BPW = B // NW
CHUNK = 1024
NCHUNK = BPW // CHUNK
NPAIR = NCHUNK // 2

assert B % NW == 0 and BPW % CHUNK == 0 and CHUNK % 8 == 0 and NCHUNK % 2 == 0

_mesh = plsc.VectorSubcoreMesh(core_axis_name="c", subcore_axis_name="s")


@functools.partial(
    pl.kernel,
    mesh=_mesh,
    out_type=jax.ShapeDtypeStruct((B, EMB_DEPTH), jnp.float32),
    scratch_types=[
        pltpu.VMEM((2, CHUNK), jnp.int32),
        pltpu.VMEM((2, CHUNK, EMB_DEPTH), jnp.float32),
        pltpu.SemaphoreType.DMA((2,)),
        pltpu.SemaphoreType.DMA((2,)),
        pltpu.SemaphoreType.DMA((2,)),
    ],
    compiler_params=pltpu.CompilerParams(use_tc_tiling_on_sc=False),
)
def _embed(x_hbm, table_hbm, out_hbm, idx_v, rows_v, si, sg, so):
    wid = lax.axis_index("s") * NC + lax.axis_index("c")
    base = wid * BPW

    def off(g):
        return base + g * CHUNK

    def idx_start(g, b):
        pltpu.async_copy(x_hbm.at[pl.ds(off(g), CHUNK)], idx_v.at[b], si.at[b])

    def idx_wait(b):
        pltpu.make_async_copy(
            x_hbm.at[pl.ds(base, CHUNK)], idx_v.at[b], si.at[b]).wait()

    def gather_start(b):
        pltpu.async_copy(table_hbm.at[idx_v.at[b]], rows_v.at[b], sg.at[b])

    def gather_wait(b):
        pltpu.make_async_copy(
            table_hbm.at[idx_v.at[b]], rows_v.at[b], sg.at[b]).wait()

    def store_start(g, b):
        pltpu.async_copy(rows_v.at[b], out_hbm.at[pl.ds(off(g), CHUNK)], so.at[b])

    def store_wait(b):
        pltpu.make_async_copy(
            rows_v.at[b], out_hbm.at[pl.ds(base, CHUNK)], so.at[b]).wait()

    def scale_rows(b):
        @plsc.parallel_loop(0, CHUNK, 1, unroll=8)
        def _(i):
            rows_v[b, i, pl.ds(0, L)] = rows_v[b, i, pl.ds(0, L)] * SCALE
            rows_v[b, i, pl.ds(L, L)] = rows_v[b, i, pl.ds(L, L)] * SCALE

    idx_start(0, 0)
    idx_start(1, 1)
    idx_wait(0)
    gather_start(0)

    def pair_body(p, carry):
        for b in (0, 1):
            g = 2 * p + b
            o = 1 - b
            gather_wait(b)

            @pl.when(g + 2 < NCHUNK)
            def _():
                idx_start(g + 2, b)

            @pl.when(g + 1 < NCHUNK)
            def _():
                @pl.when(g >= 1)
                def _():
                    store_wait(o)
                idx_wait(o)
                gather_start(o)

            scale_rows(b)
            store_start(g, b)
        return carry

    lax.fori_loop(0, NPAIR, pair_body, 0)
    store_wait(0)
    store_wait(1)


def kernel(x, table):
    out = _embed(x.reshape(B), table)
    return out.reshape(ROWS, COLS, EMB_DEPTH)


# speedup vs baseline: 5.0092x; 1.0007x over previous
import functools
import math

import jax
import jax.numpy as jnp
from jax import lax
from jax.experimental import pallas as pl
from jax.experimental.pallas import tpu as pltpu
from jax.experimental.pallas import tpu_sc as plsc

EMB_DEPTH = 32
ROWS = 16384
COLS = 200
SCALE = math.sqrt(float(EMB_DEPTH))

_info = plsc.get_sparse_core_info()
NC = _info.num_cores
NS = _info.num_subcores
L = _info.num_lanes
NW = NC * NS
RPW = ROWS // NW
RCH = 8
NCHUNK = RPW // RCH
NPAIR = NCHUNK // 2

assert ROWS % NW == 0 and RPW % RCH == 0 and NCHUNK % 2 == 0

_mesh = plsc.VectorSubcoreMesh(core_axis_name="c", subcore_axis_name="s")


@functools.partial(
    pl.kernel,
    mesh=_mesh,
    out_type=jax.ShapeDtypeStruct((ROWS, COLS, EMB_DEPTH), jnp.float32),
    scratch_types=[
        pltpu.VMEM((2, RCH, COLS), jnp.int32),
        pltpu.VMEM((2, RCH, COLS, EMB_DEPTH), jnp.float32),
        pltpu.SemaphoreType.DMA((2,)),
        pltpu.SemaphoreType.DMA((2,)),
        pltpu.SemaphoreType.DMA((2,)),
    ],
    compiler_params=pltpu.CompilerParams(use_tc_tiling_on_sc=False),
)
def _embed(x_hbm, table_hbm, out_hbm, idx_v, rows_v, si, sg, so):
    wid = lax.axis_index("s") * NC + lax.axis_index("c")
    rbase = wid * RPW

    def r0(g):
        return rbase + g * RCH

    def idx_start(g, b):
        pltpu.async_copy(x_hbm.at[pl.ds(r0(g), RCH), :], idx_v.at[b], si.at[b])

    def idx_wait(b):
        pltpu.make_async_copy(
            x_hbm.at[pl.ds(0, RCH), :], idx_v.at[b], si.at[b]).wait()

    def gather_start(b):
        for r in range(RCH):
            pltpu.async_copy(
                table_hbm.at[idx_v.at[b, r]], rows_v.at[b, r], sg.at[b])

    def gather_wait(b):
        for r in range(RCH):
            pltpu.make_async_copy(
                table_hbm.at[idx_v.at[b, r]], rows_v.at[b, r], sg.at[b]).wait()

    def store_start(g, b):
        pltpu.async_copy(rows_v.at[b], out_hbm.at[pl.ds(r0(g), RCH)], so.at[b])

    def store_wait(b):
        pltpu.make_async_copy(
            rows_v.at[b], out_hbm.at[pl.ds(0, RCH)], so.at[b]).wait()

    def scale_rows(b):
        for r in range(RCH):
            @plsc.parallel_loop(0, COLS, 1, unroll=8)
            def _(c):
                rows_v[b, r, c, pl.ds(0, L)] = rows_v[b, r, c, pl.ds(0, L)] * SCALE
                rows_v[b, r, c, pl.ds(L, L)] = rows_v[b, r, c, pl.ds(L, L)] * SCALE

    idx_start(0, 0)
    idx_start(1, 1)
    idx_wait(0)
    gather_start(0)

    def pair_body(p, carry):
        for b in (0, 1):
            g = 2 * p + b
            o = 1 - b
            gather_wait(b)

            @pl.when(g + 2 < NCHUNK)
            def _():
                idx_start(g + 2, b)

            @pl.when(g + 1 < NCHUNK)
            def _():
                @pl.when(g >= 1)
                def _():
                    store_wait(o)
                idx_wait(o)
                gather_start(o)

            scale_rows(b)
            store_start(g, b)
        return carry

    lax.fori_loop(0, NPAIR, pair_body, 0)
    store_wait(0)
    store_wait(1)


def kernel(x, table):
    return _embed(x, table)


# speedup vs baseline: 8.8687x; 1.7705x over previous
import functools
import math

import jax
import jax.numpy as jnp
from jax import lax
from jax.experimental import pallas as pl
from jax.experimental.pallas import tpu as pltpu
from jax.experimental.pallas import tpu_sc as plsc

EMB_DEPTH = 32
ROWS = 16384
COLS = 200
SCALE = math.sqrt(float(EMB_DEPTH))

_info = plsc.get_sparse_core_info()
NC = _info.num_cores
NS = _info.num_subcores
L = _info.num_lanes
NW = NC * NS
RPW = ROWS // NW
RCH = 8
NCHUNK = RPW // RCH
NPAIR = NCHUNK // 2

assert ROWS % NW == 0 and RPW % RCH == 0 and NCHUNK % 2 == 0

_mesh = plsc.VectorSubcoreMesh(core_axis_name="c", subcore_axis_name="s")


@functools.partial(
    pl.kernel,
    mesh=_mesh,
    out_type=jax.ShapeDtypeStruct((ROWS, COLS, 128), jnp.float32),
    scratch_types=[
        pltpu.VMEM((2, RCH, COLS), jnp.int32),
        pltpu.VMEM((2, RCH, COLS, EMB_DEPTH), jnp.float32),
        pltpu.SemaphoreType.DMA((2,)),
        pltpu.SemaphoreType.DMA((2,)),
        pltpu.SemaphoreType.DMA((2,)),
    ],
    compiler_params=pltpu.CompilerParams(use_tc_tiling_on_sc=False),
)
def _embed(x_hbm, table_hbm, out_hbm, idx_v, rows_v, si, sg, so):
    wid = lax.axis_index("s") * NC + lax.axis_index("c")
    rbase = wid * RPW

    def r0(g):
        return rbase + g * RCH

    def idx_start(g, b):
        pltpu.async_copy(x_hbm.at[pl.ds(r0(g), RCH), :], idx_v.at[b], si.at[b])

    def idx_wait(b):
        pltpu.make_async_copy(
            x_hbm.at[pl.ds(0, RCH), :], idx_v.at[b], si.at[b]).wait()

    def gather_start(b):
        for r in range(RCH):
            pltpu.async_copy(
                table_hbm.at[idx_v.at[b, r]], rows_v.at[b, r], sg.at[b])

    def gather_wait(b):
        for r in range(RCH):
            pltpu.make_async_copy(
                table_hbm.at[idx_v.at[b, r]], rows_v.at[b, r], sg.at[b]).wait()

    def store_start(g, b):
        pltpu.async_copy(
            rows_v.at[b],
            out_hbm.at[pl.ds(r0(g), RCH), :, pl.ds(0, EMB_DEPTH)], so.at[b])

    def store_wait(b):
        pltpu.make_async_copy(
            rows_v.at[b],
            out_hbm.at[pl.ds(0, RCH), :, pl.ds(0, EMB_DEPTH)], so.at[b]).wait()

    def scale_rows(b):
        for r in range(RCH):
            @plsc.parallel_loop(0, COLS, 1, unroll=8)
            def _(c):
                rows_v[b, r, c, pl.ds(0, L)] = rows_v[b, r, c, pl.ds(0, L)] * SCALE
                rows_v[b, r, c, pl.ds(L, L)] = rows_v[b, r, c, pl.ds(L, L)] * SCALE

    idx_start(0, 0)
    idx_start(1, 1)
    idx_wait(0)
    gather_start(0)

    def pair_body(p, carry):
        for b in (0, 1):
            g = 2 * p + b
            o = 1 - b
            gather_wait(b)

            @pl.when(g + 2 < NCHUNK)
            def _():
                idx_start(g + 2, b)

            @pl.when(g + 1 < NCHUNK)
            def _():
                @pl.when(g >= 1)
                def _():
                    store_wait(o)
                idx_wait(o)
                gather_start(o)

            scale_rows(b)
            store_start(g, b)
        return carry

    lax.fori_loop(0, NPAIR, pair_body, 0)
    store_wait(0)
    store_wait(1)


def kernel(x, table):
    return _embed(x, table)[..., :EMB_DEPTH]
